# baseline (device time: 15321 ns/iter reference)
import jax
import jax.numpy as jnp
from jax import lax
from jax.experimental import pallas as pl
from jax.experimental.pallas import tpu as pltpu

N_DEV = 4
NC = 2


def kernel(A, B):
    m, _ = A.shape
    _, n = B.shape
    w = n // NC

    def body(a_ref, b_ref, out_ref, part_ref, sum1_ref, recv1_ref, recv2_ref,
             send_sems, recv_sems):
        my_pos = lax.axis_index("i")
        pa = my_pos ^ 1
        pb = 3 - my_pos

        barrier_sem = pltpu.get_barrier_semaphore()
        for nbr in (pa, pb):
            pl.semaphore_signal(
                barrier_sem, inc=1,
                device_id=(nbr,), device_id_type=pl.DeviceIdType.MESH,
            )

        def exchange(src, q, stage, recv, target):
            return pltpu.make_async_remote_copy(
                src_ref=src.at[q],
                dst_ref=recv.at[q],
                send_sem=send_sems.at[stage, q],
                recv_sem=recv_sems.at[stage, q],
                device_id=(target,),
                device_id_type=pl.DeviceIdType.MESH,
            )

        order = [q for pair in zip(range(NC // 2), range(NC // 2, NC))
                 for q in pair]
        p1 = {q: pa if q < NC // 2 else pb for q in range(NC)}
        p2 = {q: pb if q < NC // 2 else pa for q in range(NC)}

        a_bf = a_ref[...].astype(jnp.bfloat16)
        s1 = {}
        for idx, q in enumerate(order):
            part_ref[q] = jnp.dot(
                a_bf, b_ref[:, q * w:(q + 1) * w].astype(jnp.bfloat16),
                preferred_element_type=jnp.float32,
            ).astype(jnp.bfloat16)
            if idx == 0:
                pl.semaphore_wait(barrier_sem, 2)
            s1[q] = exchange(part_ref, q, 0, recv1_ref, p1[q])
            s1[q].start()

        s2 = {}
        for q in order:
            s1[q].wait_recv()
            sum1_ref[q] = part_ref[q] + recv1_ref[q]
            s2[q] = exchange(sum1_ref, q, 1, recv2_ref, p2[q])
            s2[q].start()

        for q in order:
            s2[q].wait_recv()
            out_ref[:, q * w:(q + 1) * w] = (
                sum1_ref[q].astype(jnp.float32)
                + recv2_ref[q].astype(jnp.float32)
            )

        for q in order:
            s1[q].wait_send()
            s2[q].wait_send()

    return pl.pallas_call(
        body,
        out_shape=jax.ShapeDtypeStruct((m, n), jnp.float32),
        in_specs=[
            pl.BlockSpec(memory_space=pltpu.VMEM),
            pl.BlockSpec(memory_space=pltpu.VMEM),
        ],
        out_specs=pl.BlockSpec(memory_space=pltpu.VMEM),
        scratch_shapes=[
            pltpu.VMEM((NC, m, w), jnp.bfloat16),
            pltpu.VMEM((NC, m, w), jnp.bfloat16),
            pltpu.VMEM((NC, m, w), jnp.bfloat16),
            pltpu.VMEM((NC, m, w), jnp.bfloat16),
            pltpu.SemaphoreType.DMA((2, NC)),
            pltpu.SemaphoreType.DMA((2, NC)),
        ],
        compiler_params=pltpu.CompilerParams(collective_id=0),
    )(A, B)


# device time: 14094 ns/iter; 1.0871x vs baseline; 1.0871x over previous
import jax
import jax.numpy as jnp
from jax import lax
from jax.experimental import pallas as pl
from jax.experimental.pallas import tpu as pltpu

N_DEV = 4
NC = 4


def kernel(A, B):
    m, _ = A.shape
    _, n = B.shape
    w = n // NC

    def body(a_ref, b_ref, out_ref, part_ref, sum1_ref, recv1_ref, recv2_ref,
             send_sems, recv_sems):
        my_pos = lax.axis_index("i")
        pa = my_pos ^ 1
        pb = 3 - my_pos

        barrier_sem = pltpu.get_barrier_semaphore()
        for nbr in (pa, pb):
            pl.semaphore_signal(
                barrier_sem, inc=1,
                device_id=(nbr,), device_id_type=pl.DeviceIdType.MESH,
            )

        def exchange(src, q, stage, recv, target):
            return pltpu.make_async_remote_copy(
                src_ref=src.at[q],
                dst_ref=recv.at[q],
                send_sem=send_sems.at[stage, q],
                recv_sem=recv_sems.at[stage, q],
                device_id=(target,),
                device_id_type=pl.DeviceIdType.MESH,
            )

        order = [q for pair in zip(range(NC // 2), range(NC // 2, NC))
                 for q in pair]
        p1 = {q: pa if q < NC // 2 else pb for q in range(NC)}
        p2 = {q: pb if q < NC // 2 else pa for q in range(NC)}

        a_bf = a_ref[...].astype(jnp.bfloat16)
        s1 = {}
        for idx, q in enumerate(order):
            part_ref[q] = jnp.dot(
                a_bf, b_ref[:, q * w:(q + 1) * w].astype(jnp.bfloat16),
                preferred_element_type=jnp.float32,
            ).astype(jnp.bfloat16)
            if idx == 0:
                pl.semaphore_wait(barrier_sem, 2)
            s1[q] = exchange(part_ref, q, 0, recv1_ref, p1[q])
            s1[q].start()

        s2 = {}
        for q in order:
            s1[q].wait_recv()
            sum1_ref[q] = part_ref[q] + recv1_ref[q]
            s2[q] = exchange(sum1_ref, q, 1, recv2_ref, p2[q])
            s2[q].start()

        for q in order:
            s2[q].wait_recv()
            out_ref[:, q * w:(q + 1) * w] = (
                sum1_ref[q].astype(jnp.float32)
                + recv2_ref[q].astype(jnp.float32)
            )

        for q in order:
            s1[q].wait_send()
            s2[q].wait_send()

    return pl.pallas_call(
        body,
        out_shape=jax.ShapeDtypeStruct((m, n), jnp.float32),
        in_specs=[
            pl.BlockSpec(memory_space=pltpu.VMEM),
            pl.BlockSpec(memory_space=pltpu.VMEM),
        ],
        out_specs=pl.BlockSpec(memory_space=pltpu.VMEM),
        scratch_shapes=[
            pltpu.VMEM((NC, m, w), jnp.bfloat16),
            pltpu.VMEM((NC, m, w), jnp.bfloat16),
            pltpu.VMEM((NC, m, w), jnp.bfloat16),
            pltpu.VMEM((NC, m, w), jnp.bfloat16),
            pltpu.SemaphoreType.DMA((2, NC)),
            pltpu.SemaphoreType.DMA((2, NC)),
        ],
        compiler_params=pltpu.CompilerParams(collective_id=0),
    )(A, B)
